# baseline (speedup 1.0000x reference)
import functools

import jax
import jax.numpy as jnp
from jax import lax
from jax.experimental import pallas as pl
from jax.experimental.pallas import tpu as pltpu
from jax.experimental.pallas import tpu_sc as plsc

N = 32768
NUM_MIX = 8
NCH = 2 * NUM_MIX
NC = 2
NS = 16
LANES = 16
NW = NC * NS
CHUNK = N // NW
NSLICE = CHUNK // LANES


def _sc_body(obs_hbm, z_hbm, ct2_hbm, a2_hbm, w2_hbm, b2_hbm, w3_hbm, b3_hbm,
             mean_hbm, std_hbm,
             obs_v, z_v, ct2_v, a2_v, w2_v, b2_v, w3_v, b3_v, om_v, os_v):
    wid = lax.axis_index("c") * NS + lax.axis_index("s")
    base = wid * CHUNK

    pltpu.sync_copy(obs_hbm.at[pl.ds(base, CHUNK)], obs_v)
    pltpu.sync_copy(z_hbm.at[pl.ds(base, CHUNK)], z_v)
    pltpu.sync_copy(ct2_hbm, ct2_v)
    pltpu.sync_copy(a2_hbm, a2_v)
    pltpu.sync_copy(w2_hbm, w2_v)
    pltpu.sync_copy(b2_hbm, b2_v)
    pltpu.sync_copy(w3_hbm, w3_v)
    pltpu.sync_copy(b3_hbm, b3_v)

    def row(ref, r):
        return ref[pl.ds(r * LANES, LANES)]

    @plsc.parallel_loop(0, NSLICE, unroll=2)
    def slice_body(s):
        o = s * LANES
        obs16 = obs_v[pl.ds(o, LANES)]
        z16 = z_v[pl.ds(o, LANES)] * NCH

        h1 = []
        for j in range(NCH):
            cz = plsc.load_gather(ct2_v, [z16 + j])
            t = jnp.exp(obs16 * row(a2_v, j) + cz)
            h1.append((t - 1.0) / (t + 1.0))

        macc = row(b3_v, 0)
        sacc = row(b3_v, 1)
        for i in range(NCH):
            acc = row(b2_v, i)
            off = (i // NUM_MIX) * NUM_MIX
            for j in range(NUM_MIX):
                acc = acc + row(w2_v, i * NUM_MIX + j) * h1[off + j]
            t = jnp.exp(acc)
            h2 = (t - 1.0) / (t + 1.0)
            if i < NUM_MIX:
                macc = macc + row(w3_v, i) * h2
            else:
                sacc = sacc + row(w3_v, i) * h2

        om_v[pl.ds(o, LANES)] = macc
        os_v[pl.ds(o, LANES)] = jnp.exp(sacc)

    pltpu.sync_copy(om_v, mean_hbm.at[pl.ds(base, CHUNK)])
    pltpu.sync_copy(os_v, std_hbm.at[pl.ds(base, CHUNK)])


def _scratch_types():
    return [
        pltpu.VMEM((CHUNK,), jnp.float32),
        pltpu.VMEM((CHUNK,), jnp.int32),
        pltpu.VMEM((NUM_MIX * NCH,), jnp.float32),
        pltpu.VMEM((NCH * LANES,), jnp.float32),
        pltpu.VMEM((NCH * NUM_MIX * LANES,), jnp.float32),
        pltpu.VMEM((NCH * LANES,), jnp.float32),
        pltpu.VMEM((NCH * LANES,), jnp.float32),
        pltpu.VMEM((2 * LANES,), jnp.float32),
        pltpu.VMEM((CHUNK,), jnp.float32),
        pltpu.VMEM((CHUNK,), jnp.float32),
    ]


@functools.cache
def _sc_call():
    return functools.partial(
        pl.kernel,
        out_type=(
            jax.ShapeDtypeStruct((N,), jnp.float32),
            jax.ShapeDtypeStruct((N,), jnp.float32),
        ),
        mesh=plsc.VectorSubcoreMesh(
            core_axis_name="c", subcore_axis_name="s",
            num_cores=NC, num_subcores=NS,
        ),
        scratch_types=_scratch_types(),
        compiler_params=pltpu.CompilerParams(needs_layout_passes=False),
    )(_sc_body)


def kernel(obs, k, z, mW1, mb1, mW2, mb2, mW3, mb3,
           sW1, sb1, sW2, sb2, sW3, sb3):
    del k
    a2 = 2.0 * jnp.concatenate([mW1[:, 0], sW1[:, 0]])
    ct2 = (2.0 * jnp.concatenate(
        [mW1[:, 1:].T + mb1[None, :], sW1[:, 1:].T + sb1[None, :]], axis=1
    )).reshape(-1)
    w2 = (2.0 * jnp.concatenate([mW2, sW2], axis=0)).reshape(-1)
    b2 = 2.0 * jnp.concatenate([mb2, sb2])
    w3 = jnp.concatenate([mW3[0], sW3[0]])
    b3 = jnp.concatenate([mb3, sb3])
    mean, std = _sc_call()(
        obs, z.astype(jnp.int32), ct2,
        jnp.repeat(a2, LANES), jnp.repeat(w2, LANES),
        jnp.repeat(b2, LANES), jnp.repeat(w3, LANES),
        jnp.repeat(b3, LANES),
    )
    return mean, std


# speedup vs baseline: 1.0223x; 1.0223x over previous
import functools

import jax
import jax.numpy as jnp
from jax import lax
from jax.experimental import pallas as pl
from jax.experimental.pallas import tpu as pltpu
from jax.experimental.pallas import tpu_sc as plsc

N = 32768
NUM_MIX = 8
NCH = 2 * NUM_MIX
NC = 2
NS = 16
LANES = 16
NW = NC * NS
CHUNK = N // NW
NSLICE = CHUNK // LANES


def _sc_body(obs_hbm, z_hbm, ct2_hbm, a2_hbm, w2_hbm, b2_hbm, w3_hbm, b3_hbm,
             mean_hbm, std_hbm,
             obs_v, z_v, ct2_v, a2_v, w2_v, b2_v, w3_v, b3_v, om_v, os_v):
    wid = lax.axis_index("c") * NS + lax.axis_index("s")
    base = wid * CHUNK

    pltpu.sync_copy(obs_hbm.at[pl.ds(base, CHUNK)], obs_v)
    pltpu.sync_copy(z_hbm.at[pl.ds(base, CHUNK)], z_v)
    pltpu.sync_copy(ct2_hbm, ct2_v)
    pltpu.sync_copy(a2_hbm, a2_v)
    pltpu.sync_copy(w2_hbm, w2_v)
    pltpu.sync_copy(b2_hbm, b2_v)
    pltpu.sync_copy(w3_hbm, w3_v)
    pltpu.sync_copy(b3_hbm, b3_v)

    def row(ref, r):
        return ref[pl.ds(r * LANES, LANES)]

    def tree_sum(terms):
        while len(terms) > 1:
            terms = [terms[p] + terms[p + 1] for p in range(0, len(terms) - 1, 2)] + (
                [terms[-1]] if len(terms) % 2 else []
            )
        return terms[0]

    @plsc.parallel_loop(0, NSLICE, unroll=1)
    def slice_body(s):
        o = s * LANES
        obs16 = obs_v[pl.ds(o, LANES)]
        z16 = z_v[pl.ds(o, LANES)] * NCH

        h1 = []
        for j in range(NCH):
            cz = plsc.load_gather(ct2_v, [z16 + j])
            t = jnp.exp(obs16 * row(a2_v, j) + cz)
            h1.append((t - 1.0) / (t + 1.0))

        h2 = []
        for i in range(NCH):
            off = (i // NUM_MIX) * NUM_MIX
            terms = [row(w2_v, i * NUM_MIX + j) * h1[off + j]
                     for j in range(NUM_MIX)]
            t = jnp.exp(tree_sum(terms) + row(b2_v, i))
            h2.append((t - 1.0) / (t + 1.0))

        macc = tree_sum([row(w3_v, i) * h2[i] for i in range(NUM_MIX)])
        sacc = tree_sum([row(w3_v, NUM_MIX + i) * h2[NUM_MIX + i]
                         for i in range(NUM_MIX)])
        om_v[pl.ds(o, LANES)] = macc + row(b3_v, 0)
        os_v[pl.ds(o, LANES)] = jnp.exp(sacc + row(b3_v, 1))

    pltpu.sync_copy(om_v, mean_hbm.at[pl.ds(base, CHUNK)])
    pltpu.sync_copy(os_v, std_hbm.at[pl.ds(base, CHUNK)])


def _scratch_types():
    return [
        pltpu.VMEM((CHUNK,), jnp.float32),
        pltpu.VMEM((CHUNK,), jnp.int32),
        pltpu.VMEM((NUM_MIX * NCH,), jnp.float32),
        pltpu.VMEM((NCH * LANES,), jnp.float32),
        pltpu.VMEM((NCH * NUM_MIX * LANES,), jnp.float32),
        pltpu.VMEM((NCH * LANES,), jnp.float32),
        pltpu.VMEM((NCH * LANES,), jnp.float32),
        pltpu.VMEM((2 * LANES,), jnp.float32),
        pltpu.VMEM((CHUNK,), jnp.float32),
        pltpu.VMEM((CHUNK,), jnp.float32),
    ]


@functools.cache
def _sc_call():
    return functools.partial(
        pl.kernel,
        out_type=(
            jax.ShapeDtypeStruct((N,), jnp.float32),
            jax.ShapeDtypeStruct((N,), jnp.float32),
        ),
        mesh=plsc.VectorSubcoreMesh(
            core_axis_name="c", subcore_axis_name="s",
            num_cores=NC, num_subcores=NS,
        ),
        scratch_types=_scratch_types(),
        compiler_params=pltpu.CompilerParams(needs_layout_passes=False),
    )(_sc_body)


def kernel(obs, k, z, mW1, mb1, mW2, mb2, mW3, mb3,
           sW1, sb1, sW2, sb2, sW3, sb3):
    del k
    a2 = 2.0 * jnp.concatenate([mW1[:, 0], sW1[:, 0]])
    ct2 = (2.0 * jnp.concatenate(
        [mW1[:, 1:].T + mb1[None, :], sW1[:, 1:].T + sb1[None, :]], axis=1
    )).reshape(-1)
    w2 = (2.0 * jnp.concatenate([mW2, sW2], axis=0)).reshape(-1)
    b2 = 2.0 * jnp.concatenate([mb2, sb2])
    w3 = jnp.concatenate([mW3[0], sW3[0]])
    b3 = jnp.concatenate([mb3, sb3])
    mean, std = _sc_call()(
        obs, z.astype(jnp.int32), ct2,
        jnp.repeat(a2, LANES), jnp.repeat(w2, LANES),
        jnp.repeat(b2, LANES), jnp.repeat(w3, LANES),
        jnp.repeat(b3, LANES),
    )
    return mean, std


# speedup vs baseline: 1.0985x; 1.0745x over previous
import functools

import jax
import jax.numpy as jnp
from jax import lax
from jax.experimental import pallas as pl
from jax.experimental.pallas import tpu as pltpu
from jax.experimental.pallas import tpu_sc as plsc

N = 32768
NUM_MIX = 8
NC = 2
NS = 16
LANES = 16
NW = NC * NS
CHUNK = N // NW
NSLICE = CHUNK // LANES
U = 4
NGROUP = NSLICE // U

_A_OFF = 0
_W2_OFF = 8
_B2_OFF = _W2_OFF + 64
_W3_OFF = _B2_OFF + 8
_B3_OFF = _W3_OFF + 8
_WP_ROWS = _B3_OFF + 1


def _sc_body(obs_hbm, z_hbm, ctm_hbm, cts_hbm, wpm_hbm, wps_hbm,
             mean_hbm, std_hbm,
             obs_v, z_v, ctm_v, cts_v, wpm_v, wps_v, om_v, os_v):
    wid = lax.axis_index("c") * NS + lax.axis_index("s")
    base = wid * CHUNK

    pltpu.sync_copy(obs_hbm.at[pl.ds(base, CHUNK)], obs_v)
    pltpu.sync_copy(z_hbm.at[pl.ds(base, CHUNK)], z_v)
    pltpu.sync_copy(ctm_hbm, ctm_v)
    pltpu.sync_copy(cts_hbm, cts_v)
    pltpu.sync_copy(wpm_hbm, wpm_v)
    pltpu.sync_copy(wps_hbm, wps_v)

    def run_net(ct_v, wp_v, out_v, is_std):
        def row(r):
            return wp_v[pl.ds(r * LANES, LANES)]

        @plsc.parallel_loop(0, NGROUP, unroll=1)
        def group_body(g):
            o0 = g * (U * LANES)
            obs16 = []
            z8 = []
            for u in range(U):
                o = o0 + u * LANES
                obs16.append(obs_v[pl.ds(o, LANES)])
                z8.append(z_v[pl.ds(o, LANES)] * NUM_MIX)

            h1 = [[None] * NUM_MIX for _ in range(U)]
            for j in range(NUM_MIX):
                aj = row(_A_OFF + j)
                for u in range(U):
                    cz = plsc.load_gather(ct_v, [z8[u] + j])
                    t = jnp.exp(obs16[u] * aj + cz)
                    h1[u][j] = (t - 1.0) / (t + 1.0)

            out = [row(_B3_OFF)] * U
            for i in range(NUM_MIX):
                acc = [row(_B2_OFF + i)] * U
                for j in range(NUM_MIX):
                    w = row(_W2_OFF + i * NUM_MIX + j)
                    for u in range(U):
                        acc[u] = acc[u] + w * h1[u][j]
                w3 = row(_W3_OFF + i)
                for u in range(U):
                    t = jnp.exp(acc[u])
                    out[u] = out[u] + w3 * ((t - 1.0) / (t + 1.0))

            for u in range(U):
                o = o0 + u * LANES
                out_v[pl.ds(o, LANES)] = jnp.exp(out[u]) if is_std else out[u]

    run_net(ctm_v, wpm_v, om_v, False)
    run_net(cts_v, wps_v, os_v, True)

    pltpu.sync_copy(om_v, mean_hbm.at[pl.ds(base, CHUNK)])
    pltpu.sync_copy(os_v, std_hbm.at[pl.ds(base, CHUNK)])


def _scratch_types():
    return [
        pltpu.VMEM((CHUNK,), jnp.float32),
        pltpu.VMEM((CHUNK,), jnp.int32),
        pltpu.VMEM((NUM_MIX * NUM_MIX,), jnp.float32),
        pltpu.VMEM((NUM_MIX * NUM_MIX,), jnp.float32),
        pltpu.VMEM((_WP_ROWS * LANES,), jnp.float32),
        pltpu.VMEM((_WP_ROWS * LANES,), jnp.float32),
        pltpu.VMEM((CHUNK,), jnp.float32),
        pltpu.VMEM((CHUNK,), jnp.float32),
    ]


@functools.cache
def _sc_call():
    return functools.partial(
        pl.kernel,
        out_type=(
            jax.ShapeDtypeStruct((N,), jnp.float32),
            jax.ShapeDtypeStruct((N,), jnp.float32),
        ),
        mesh=plsc.VectorSubcoreMesh(
            core_axis_name="c", subcore_axis_name="s",
            num_cores=NC, num_subcores=NS,
        ),
        scratch_types=_scratch_types(),
        compiler_params=pltpu.CompilerParams(needs_layout_passes=False),
    )(_sc_body)


def _pack_net(W1, b1, W2, b2, W3, b3):
    ct = (2.0 * (W1[:, 1:].T + b1[None, :])).reshape(-1)
    wp = jnp.concatenate([
        jnp.repeat(2.0 * W1[:, 0], LANES),
        jnp.repeat(2.0 * W2.reshape(-1), LANES),
        jnp.repeat(2.0 * b2, LANES),
        jnp.repeat(W3[0], LANES),
        jnp.repeat(b3, LANES),
    ])
    return ct, wp


def kernel(obs, k, z, mW1, mb1, mW2, mb2, mW3, mb3,
           sW1, sb1, sW2, sb2, sW3, sb3):
    del k
    ctm, wpm = _pack_net(mW1, mb1, mW2, mb2, mW3, mb3)
    cts, wps = _pack_net(sW1, sb1, sW2, sb2, sW3, sb3)
    mean, std = _sc_call()(obs, z.astype(jnp.int32), ctm, cts, wpm, wps)
    return mean, std
